# Initial kernel scaffold; baseline (speedup 1.0000x reference)
import jax
import jax.numpy as jnp
from jax.experimental import pallas as pl


def kernel(numerical_input, categorical_inputs, emb_tables, bw0, bb0, bw1, bb1, bw2, bb2, tw0, tb0, tw1, tb1, tw2, tb2, tw3, tb3, tw4, tb4):
    raise NotImplementedError("write your pallas kernel here")


# speedup vs baseline: 6.1597x; 6.1597x over previous
import functools

import jax
import jax.numpy as jnp
import numpy as np
from jax import lax
from jax.experimental import pallas as pl
from jax.experimental.pallas import tpu as pltpu
from jax.experimental.pallas import tpu_sc as plsc

B = 16384
NUM = 13
F = 26
V = 100000
D = 32
NF = F + 1
NPAIR = NF * (NF - 1) // 2

_NC, _NS = 2, 16
NW = _NC * _NS
ROWS = B * F
ROWS_W = ROWS // NW
CH = 128
NCH = ROWS_W // CH

@functools.lru_cache(maxsize=None)
def _make_sc_gather():
    mesh = plsc.VectorSubcoreMesh(core_axis_name="c", subcore_axis_name="s")
    return functools.partial(
        pl.kernel,
        out_type=jax.ShapeDtypeStruct((ROWS, D), jnp.float32),
        mesh=mesh,
        scratch_types=[
            pltpu.VMEM((NCH, CH), jnp.int32),
            pltpu.VMEM((CH, D), jnp.float32),
            pltpu.VMEM((CH, D), jnp.float32),
            pltpu.SemaphoreType.DMA,
            pltpu.SemaphoreType.DMA,
        ],
        compiler_params=pltpu.CompilerParams(use_tc_tiling_on_sc=False),
    )(_sc_gather_body)


def _sc_gather_body(table_hbm, idx_hbm, out_hbm, idx_v, buf0, buf1, sem0, sem1):
    wid = lax.axis_index("s") * _NC + lax.axis_index("c")
    base = wid * ROWS_W
    pltpu.sync_copy(idx_hbm.at[wid], idx_v)
    pltpu.async_copy(table_hbm.at[idx_v.at[0]], buf0, sem0)

    def body(g, carry):
        j0 = 2 * g
        pltpu.async_copy(table_hbm.at[idx_v.at[j0 + 1]], buf1, sem1)
        pltpu.make_async_copy(table_hbm.at[idx_v.at[j0]], buf0, sem0).wait()
        pltpu.sync_copy(buf0, out_hbm.at[pl.ds(base + j0 * CH, CH)])

        @pl.when(j0 + 2 < NCH)
        def _():
            pltpu.async_copy(table_hbm.at[idx_v.at[j0 + 2]], buf0, sem0)

        pltpu.make_async_copy(table_hbm.at[idx_v.at[j0 + 1]], buf1, sem1).wait()
        pltpu.sync_copy(buf1, out_hbm.at[pl.ds(base + (j0 + 1) * CH, CH)])
        return carry

    lax.fori_loop(0, NCH // 2, body, 0)


BK = 512


def _build_consts():
    p0 = np.zeros((D, 32 * D), np.float32)
    for d in range(D):
        p0[d, d * 32 + 0] = 1.0
    p1 = np.zeros((F * D, 32 * D), np.float32)
    for f in range(F):
        for d in range(D):
            p1[f * D + d, d * 32 + 1 + f] = 1.0
    li, lj = np.tril_indices(NF, -1)
    sl = np.zeros((NF, NPAIR), np.float32)
    sr = np.zeros((NF, NPAIR), np.float32)
    for p in range(NPAIR):
        sl[li[p], p] = 1.0
        sr[lj[p], p] = 1.0
    return jnp.asarray(p0), jnp.asarray(p1), jnp.asarray(sl), jnp.asarray(sr)


def _tc_body(num_ref, embs_ref, p0_ref, p1_ref, sl_ref, sr_ref,
             bw0_ref, bb0_ref, bw1_ref, bb1_ref, bw2_ref, bb2_ref,
             tw0a_ref, tw0b_ref, tb0_ref, tw1_ref, tb1_ref, tw2_ref, tb2_ref,
             tw3_ref, tb3_ref, tw4_ref, tb4_ref, out_ref):
    x = num_ref[...]
    x = jnp.maximum(x @ bw0_ref[...] + bb0_ref[...], 0.0)
    x = jnp.maximum(x @ bw1_ref[...] + bb1_ref[...], 0.0)
    bot = jnp.maximum(x @ bw2_ref[...] + bb2_ref[...], 0.0)
    e = embs_ref[...]
    t = bot @ p0_ref[...] + e @ p1_ref[...]
    sl = sl_ref[...]
    sr = sr_ref[...]
    z = jnp.zeros((BK, NPAIR), jnp.float32)
    for d in range(D):
        u = t[:, 32 * d: 32 * d + NF]
        z = z + (u @ sl) * (u @ sr)
    h = jnp.maximum(bot @ tw0a_ref[...] + z @ tw0b_ref[...] + tb0_ref[...], 0.0)
    h = jnp.maximum(h @ tw1_ref[...] + tb1_ref[...], 0.0)
    h = jnp.maximum(h @ tw2_ref[...] + tb2_ref[...], 0.0)
    h = jnp.maximum(h @ tw3_ref[...] + tb3_ref[...], 0.0)
    out_ref[...] = h @ tw4_ref[...] + tb4_ref[...]


def _full2(shape):
    return pl.BlockSpec(shape, lambda i: (0, 0))


def _tc_fused(num, embs, p0, p1, sl, sr, bw0, bb0, bw1, bb1, bw2, bb2,
              tw0a, tw0b, tb0, tw1, tb1, tw2, tb2, tw3, tb3, tw4, tb4):
    grid = (B // BK,)
    in_specs = [
        pl.BlockSpec((BK, NUM), lambda i: (i, 0)),
        pl.BlockSpec((BK, F * D), lambda i: (i, 0)),
        _full2(p0.shape), _full2(p1.shape), _full2(sl.shape), _full2(sr.shape),
        _full2(bw0.shape), _full2(bb0.shape), _full2(bw1.shape), _full2(bb1.shape),
        _full2(bw2.shape), _full2(bb2.shape),
        _full2(tw0a.shape), _full2(tw0b.shape), _full2(tb0.shape),
        _full2(tw1.shape), _full2(tb1.shape), _full2(tw2.shape), _full2(tb2.shape),
        _full2(tw3.shape), _full2(tb3.shape), _full2(tw4.shape), _full2(tb4.shape),
    ]
    return pl.pallas_call(
        _tc_body,
        grid=grid,
        in_specs=in_specs,
        out_specs=pl.BlockSpec((BK, 1), lambda i: (i, 0)),
        out_shape=jax.ShapeDtypeStruct((B, 1), jnp.float32),
    )(num, embs, p0, p1, sl, sr, bw0, bb0, bw1, bb1, bw2, bb2,
      tw0a, tw0b, tb0, tw1, tb1, tw2, tb2, tw3, tb3, tw4, tb4)


def kernel(numerical_input, categorical_inputs, emb_tables,
           bw0, bb0, bw1, bb1, bw2, bb2,
           tw0, tb0, tw1, tb1, tw2, tb2, tw3, tb3, tw4, tb4):
    cat = categorical_inputs.astype(jnp.int32)
    flat_idx = (cat + (jnp.arange(F, dtype=jnp.int32) * V)[None, :]).reshape(-1)
    idx3 = flat_idx.reshape(NW, NCH, CH)
    table = emb_tables.reshape(F * V, D)
    embs = _make_sc_gather()(table, idx3).reshape(B, F * D)

    p0, p1, sl, sr = _build_consts()
    tw0a = tw0[:D]
    tw0b = tw0[D:D + NPAIR]
    out = _tc_fused(
        numerical_input, embs, p0, p1, sl, sr,
        bw0, bb0.reshape(1, -1), bw1, bb1.reshape(1, -1), bw2, bb2.reshape(1, -1),
        tw0a, tw0b, tb0.reshape(1, -1), tw1, tb1.reshape(1, -1),
        tw2, tb2.reshape(1, -1), tw3, tb3.reshape(1, -1), tw4, tb4.reshape(1, -1))
    return out
